# Initial kernel scaffold; baseline (speedup 1.0000x reference)
import jax
import jax.numpy as jnp
from jax.experimental import pallas as pl


def kernel(seqs, d, c, u, Wd, Wc, Wu):
    raise NotImplementedError("write your pallas kernel here")


# speedup vs baseline: 3.5799x; 3.5799x over previous
import functools

import jax
import jax.numpy as jnp
from jax import lax
from jax.experimental import pallas as pl
from jax.experimental.pallas import tpu as pltpu
from jax.experimental.pallas import tpu_sc as plsc

NC, NS = 2, 16
NW = NC * NS
CH = 128


def _gather_rows(table, idx, dim):
    n = idx.shape[0]
    per_w = n // NW
    n_ch = per_w // CH
    mesh = plsc.VectorSubcoreMesh(
        core_axis_name="c", subcore_axis_name="s", num_cores=NC, num_subcores=NS
    )

    cp = pltpu.CompilerParams(use_tc_tiling_on_sc=(dim % 128 == 0))

    @functools.partial(
        pl.kernel,
        out_type=jax.ShapeDtypeStruct((n, dim), jnp.float32),
        mesh=mesh,
        compiler_params=cp,
        scratch_types=[
            pltpu.VMEM((CH,), jnp.int32),
            pltpu.VMEM((CH, dim), jnp.float32),
            pltpu.SemaphoreType.DMA,
        ],
    )
    def k(table_hbm, idx_hbm, out_hbm, idx_v, rows_v, sem):
        wid = lax.axis_index("s") * NC + lax.axis_index("c")
        w_base = wid * per_w

        @pl.loop(0, n_ch)
        def _(j):
            base = w_base + j * CH
            pltpu.sync_copy(idx_hbm.at[pl.ds(base, CH)], idx_v)
            pltpu.async_copy(table_hbm.at[idx_v], rows_v, sem).wait()
            pltpu.sync_copy(rows_v, out_hbm.at[pl.ds(base, CH)])

    return k(table, idx)


def _tc_add(a, b):
    n, dim = a.shape
    blk = 2048

    def body(a_ref, b_ref, o_ref):
        o_ref[...] = a_ref[...] + b_ref[...]

    return pl.pallas_call(
        body,
        grid=(n // blk,),
        in_specs=[
            pl.BlockSpec((blk, dim), lambda i: (i, 0)),
            pl.BlockSpec((blk, dim), lambda i: (i, 0)),
        ],
        out_specs=pl.BlockSpec((blk, dim), lambda i: (i, 0)),
        out_shape=jax.ShapeDtypeStruct((n, dim), jnp.float32),
    )(a, b)


def kernel(seqs, d, c, u, Wd, Wc, Wu):
    B, L, U = seqs.shape
    BL = B * L
    dv = Wd.shape[0]

    u2 = u.reshape(BL).astype(jnp.int32)
    dc_idx = jnp.stack(
        [d.astype(jnp.int32), c.astype(jnp.int32) + dv], axis=-1
    ).reshape(2 * BL)
    Wdc = jnp.concatenate([Wd, Wc], axis=0)

    ue = _gather_rows(Wu, u2, U)
    d_c = _gather_rows(Wdc, dc_idx, Wd.shape[1])
    seqs_u = _tc_add(seqs.reshape(BL, U), ue)

    return seqs_u.reshape(B, L, U), d_c.reshape(B, L, 2 * Wd.shape[1])
